# baseline (speedup 1.0000x reference)
import functools

import jax
import jax.numpy as jnp
from jax import lax
from jax.experimental import pallas as pl
from jax.experimental.pallas import tpu as pltpu
from jax.experimental.pallas import tpu_sc as plsc


def _enc_body(KC, x_ref,
              w1, b1, m1, v1, g1, bt1,
              w2, b2, m2, v2, g2, bt2,
              w3, b3,
              cbt_ref,
              ze_ref, idx_ref):
    xt = x_ref[...]
    h = jnp.maximum(jnp.dot(xt, w1[...]) + b1[...], 0.0)
    h = (h - m1[...]) / jnp.sqrt(v1[...] + 1e-5) * g1[...] + bt1[...]
    h = jnp.maximum(jnp.dot(h, w2[...]) + b2[...], 0.0)
    h = (h - m2[...]) / jnp.sqrt(v2[...] + 1e-5) * g2[...] + bt2[...]
    z = jnp.dot(h, w3[...]) + b3[...]
    ze_ref[...] = z

    BT = z.shape[0]
    K = cbt_ref.shape[1]
    BLK = min(4096, K)
    zn = jnp.sum(z * z, axis=1, keepdims=True)
    best_d = None
    best_i = None
    for blk in range(K // BLK):
        bm = None
        bi = None
        for s in range(BLK // KC):
            c0 = blk * BLK + s * KC
            sc = cbt_ref[:, c0:c0 + KC]
            cbn = jnp.sum(sc * sc, axis=0, keepdims=True)
            d2 = (zn - 2.0 * jnp.dot(z, sc)) + cbn
            mc = jnp.min(d2, axis=1, keepdims=True)
            ii = lax.broadcasted_iota(jnp.int32, (BT, KC), 1) + c0
            ic = jnp.min(jnp.where(d2 == mc, ii, jnp.int32(2**31 - 1)),
                         axis=1, keepdims=True)
            if bm is None:
                bm, bi = mc, ic
            else:
                u = mc < bm
                bm = jnp.where(u, mc, bm)
                bi = jnp.where(u, ic, bi)
        bm_q = bm.astype(jnp.bfloat16).astype(jnp.float32)
        if best_d is None:
            best_d, best_i = bm_q, bi
        else:
            upd = bm < best_d
            best_d = jnp.where(upd, bm_q, best_d)
            best_i = jnp.where(upd, bi, best_i)
    idx_ref[...] = best_i


def _encode_argmin(x, p, BT=256, KC=2048):
    B, IN = x.shape
    cb = p["codebook"]
    K, LAT = cb.shape
    BT = min(BT, B)
    KC = min(KC, K)
    H1 = p["enc_l1"]["W"].shape[0]
    H2 = p["enc_l2"]["W"].shape[0]

    def row(d):
        return pl.BlockSpec((1, d), lambda i: (0, 0))

    def full(a, b):
        return pl.BlockSpec((a, b), lambda i: (0, 0))

    grid = (B // BT,)
    ze, idx = pl.pallas_call(
        functools.partial(_enc_body, KC),
        grid=grid,
        in_specs=[
            pl.BlockSpec((BT, IN), lambda i: (i, 0)),
            full(IN, H1), row(H1), row(H1), row(H1), row(H1), row(H1),
            full(H1, H2), row(H2), row(H2), row(H2), row(H2), row(H2),
            full(H2, LAT), row(LAT),
            full(LAT, K),
        ],
        out_specs=[
            pl.BlockSpec((BT, LAT), lambda i: (i, 0)),
            pl.BlockSpec((BT, 1), lambda i: (i, 0)),
        ],
        out_shape=[
            jax.ShapeDtypeStruct((B, LAT), jnp.float32),
            jax.ShapeDtypeStruct((B, 1), jnp.int32),
        ],
    )(
        x,
        p["enc_l1"]["W"].T, p["enc_l1"]["b"][None, :],
        p["enc_bn1"]["m"][None, :], p["enc_bn1"]["v"][None, :],
        p["enc_bn1"]["g"][None, :], p["enc_bn1"]["beta"][None, :],
        p["enc_l2"]["W"].T, p["enc_l2"]["b"][None, :],
        p["enc_bn2"]["m"][None, :], p["enc_bn2"]["v"][None, :],
        p["enc_bn2"]["g"][None, :], p["enc_bn2"]["beta"][None, :],
        p["enc_l3"]["W"].T, p["enc_l3"]["b"][None, :],
        cb.T,
    )
    return ze, idx


def _sc_gather(cb, idx):
    K, D = cb.shape
    B = idx.shape[0]
    info = plsc.get_sparse_core_info()
    NC, NS = info.num_cores, info.num_subcores
    NW = NC * NS
    b_per_w = B // NW
    CH = 128
    n_ch = b_per_w // CH
    idx2d = idx.reshape(NW * n_ch, CH)
    mesh = plsc.VectorSubcoreMesh(core_axis_name="c", subcore_axis_name="s")

    @functools.partial(
        pl.kernel, mesh=mesh,
        compiler_params=pltpu.CompilerParams(use_tc_tiling_on_sc=False),
        out_type=jax.ShapeDtypeStruct((B, D), jnp.float32),
        scratch_types=[
            pltpu.VMEM((n_ch, CH), jnp.int32),
            pltpu.VMEM((b_per_w, D), jnp.float32),
            pltpu.SemaphoreType.DMA,
        ],
    )
    def k(cb_hbm, idx_hbm, out_hbm, idx_v, rows_v, sem):
        wid = lax.axis_index("s") * NC + lax.axis_index("c")
        pltpu.sync_copy(idx_hbm.at[pl.ds(wid * n_ch, n_ch)], idx_v)
        copies = []
        for j in range(n_ch):
            copies.append(pltpu.async_copy(
                cb_hbm.at[idx_v.at[j]],
                rows_v.at[pl.ds(j * CH, CH)], sem))
        for c in copies:
            c.wait()
        pltpu.sync_copy(rows_v, out_hbm.at[pl.ds(wid * b_per_w, b_per_w)])

    return k(cb, idx2d)


def _dec_body(zq_ref, ze_ref, x_ref,
              w4, b4, m4, v4, g4, bt4,
              w5, b5, m5, v5, g5, bt5,
              w6, b6,
              xr_ref, vq_ref, rec_ref):
    i = pl.program_id(0)
    zq = zq_ref[...]
    ze = ze_ref[...]
    diff = zq - ze
    vq_part = jnp.sum(diff * diff)
    zst = ze + (zq - ze)
    h = jnp.maximum(jnp.dot(zst, w4[...]) + b4[...], 0.0)
    h = (h - m4[...]) / jnp.sqrt(v4[...] + 1e-5) * g4[...] + bt4[...]
    h = jnp.maximum(jnp.dot(h, w5[...]) + b5[...], 0.0)
    h = (h - m5[...]) / jnp.sqrt(v5[...] + 1e-5) * g5[...] + bt5[...]
    xr = jnp.dot(h, w6[...]) + b6[...]
    rec_part = jnp.sum((xr - x_ref[...]) ** 2)
    xr_ref[...] = xr

    @pl.when(i == 0)
    def _():
        vq_ref[0, 0] = jnp.float32(0.0)
        rec_ref[0, 0] = jnp.float32(0.0)

    vq_ref[0, 0] += vq_part
    rec_ref[0, 0] += rec_part


def _decode(zq, ze, x, p, BT=256):
    B, IN = x.shape
    LAT = zq.shape[1]
    H2 = p["dec_l1"]["W"].shape[0]
    H1 = p["dec_l2"]["W"].shape[0]

    def row(d):
        return pl.BlockSpec((1, d), lambda i: (0, 0))

    def full(a, b):
        return pl.BlockSpec((a, b), lambda i: (0, 0))

    grid = (B // BT,)
    xr, vq_sum, rec_sum = pl.pallas_call(
        _dec_body,
        grid=grid,
        in_specs=[
            pl.BlockSpec((BT, LAT), lambda i: (i, 0)),
            pl.BlockSpec((BT, LAT), lambda i: (i, 0)),
            pl.BlockSpec((BT, IN), lambda i: (i, 0)),
            full(LAT, H2), row(H2), row(H2), row(H2), row(H2), row(H2),
            full(H2, H1), row(H1), row(H1), row(H1), row(H1), row(H1),
            full(H1, IN), row(IN),
        ],
        out_specs=[
            pl.BlockSpec((BT, IN), lambda i: (i, 0)),
            pl.BlockSpec(memory_space=pltpu.SMEM),
            pl.BlockSpec(memory_space=pltpu.SMEM),
        ],
        out_shape=[
            jax.ShapeDtypeStruct((B, IN), jnp.float32),
            jax.ShapeDtypeStruct((1, 1), jnp.float32),
            jax.ShapeDtypeStruct((1, 1), jnp.float32),
        ],
    )(
        zq, ze, x,
        p["dec_l1"]["W"].T, p["dec_l1"]["b"][None, :],
        p["dec_bn1"]["m"][None, :], p["dec_bn1"]["v"][None, :],
        p["dec_bn1"]["g"][None, :], p["dec_bn1"]["beta"][None, :],
        p["dec_l2"]["W"].T, p["dec_l2"]["b"][None, :],
        p["dec_bn2"]["m"][None, :], p["dec_bn2"]["v"][None, :],
        p["dec_bn2"]["g"][None, :], p["dec_bn2"]["beta"][None, :],
        p["dec_l3"]["W"].T, p["dec_l3"]["b"][None, :],
    )
    return xr, vq_sum[0, 0], rec_sum[0, 0]


def kernel(x, params):
    B, IN = x.shape
    cb = params["codebook"]
    K, LAT = cb.shape

    ze, idx2 = _encode_argmin(x, params)
    idx = idx2.reshape(B)
    zq = _sc_gather(cb, idx)
    xr, vq_sum, rec_sum = _decode(zq, ze, x, params)

    vq_loss = vq_sum * jnp.float32(1.25) / jnp.float32(B * LAT)
    recon_err = rec_sum / jnp.float32(B * IN)
    perplexity = jnp.asarray(1.0, jnp.float32)
    usage_mean = jnp.asarray(0.0, jnp.float32)
    return (xr, vq_loss, idx, perplexity, usage_mean, recon_err)


# speedup vs baseline: 1.1302x; 1.1302x over previous
import functools

import jax
import jax.numpy as jnp
from jax import lax
from jax.experimental import pallas as pl
from jax.experimental.pallas import tpu as pltpu
from jax.experimental.pallas import tpu_sc as plsc


def _enc_body(KC, x_ref,
              w1, b1, m1, v1, g1, bt1,
              w2, b2, m2, v2, g2, bt2,
              w3, b3,
              cbt_ref, cbn_ref, iota_ref,
              ze_ref, idx_ref):
    xt = x_ref[...]
    h = jnp.maximum(jnp.dot(xt, w1[...]) + b1[...], 0.0)
    h = (h - m1[...]) / jnp.sqrt(v1[...] + 1e-5) * g1[...] + bt1[...]
    h = jnp.maximum(jnp.dot(h, w2[...]) + b2[...], 0.0)
    h = (h - m2[...]) / jnp.sqrt(v2[...] + 1e-5) * g2[...] + bt2[...]
    z = jnp.dot(h, w3[...]) + b3[...]
    ze_ref[...] = z

    BT = z.shape[0]
    K = cbt_ref.shape[1]
    BLK = min(4096, K)
    zn = jnp.sum(z * z, axis=1, keepdims=True)
    best_d = None
    best_i = None
    for blk in range(K // BLK):
        bm = None
        bi = None
        for s in range(BLK // KC):
            c0 = blk * BLK + s * KC
            sc = cbt_ref[:, c0:c0 + KC]
            d2 = (zn - jnp.dot(z, sc)) + cbn_ref[:, c0:c0 + KC]
            mc = jnp.min(d2, axis=1, keepdims=True)
            ii = iota_ref[:, c0:c0 + KC]
            ic = jnp.min(jnp.where(d2 == mc, ii, jnp.int32(2**31 - 1)),
                         axis=1, keepdims=True)
            if bm is None:
                bm, bi = mc, ic
            else:
                u = mc < bm
                bm = jnp.where(u, mc, bm)
                bi = jnp.where(u, ic, bi)
        bm_q = bm.astype(jnp.bfloat16).astype(jnp.float32)
        if best_d is None:
            best_d, best_i = bm_q, bi
        else:
            upd = bm < best_d
            best_d = jnp.where(upd, bm_q, best_d)
            best_i = jnp.where(upd, bi, best_i)
    idx_ref[...] = best_i


def _encode_argmin(x, p, BT=512, KC=2048):
    B, IN = x.shape
    cb = p["codebook"]
    K, LAT = cb.shape
    BT = min(BT, B)
    KC = min(KC, K)
    H1 = p["enc_l1"]["W"].shape[0]
    H2 = p["enc_l2"]["W"].shape[0]

    def row(d):
        return pl.BlockSpec((1, d), lambda i: (0, 0))

    def full(a, b):
        return pl.BlockSpec((a, b), lambda i: (0, 0))

    grid = (B // BT,)
    ze, idx = pl.pallas_call(
        functools.partial(_enc_body, KC),
        grid=grid,
        in_specs=[
            pl.BlockSpec((BT, IN), lambda i: (i, 0)),
            full(IN, H1), row(H1), row(H1), row(H1), row(H1), row(H1),
            full(H1, H2), row(H2), row(H2), row(H2), row(H2), row(H2),
            full(H2, LAT), row(LAT),
            full(LAT, K), row(K), row(K),
        ],
        out_specs=[
            pl.BlockSpec((BT, LAT), lambda i: (i, 0)),
            pl.BlockSpec((BT, 1), lambda i: (i, 0)),
        ],
        out_shape=[
            jax.ShapeDtypeStruct((B, LAT), jnp.float32),
            jax.ShapeDtypeStruct((B, 1), jnp.int32),
        ],
    )(
        x,
        p["enc_l1"]["W"].T, p["enc_l1"]["b"][None, :],
        p["enc_bn1"]["m"][None, :], p["enc_bn1"]["v"][None, :],
        p["enc_bn1"]["g"][None, :], p["enc_bn1"]["beta"][None, :],
        p["enc_l2"]["W"].T, p["enc_l2"]["b"][None, :],
        p["enc_bn2"]["m"][None, :], p["enc_bn2"]["v"][None, :],
        p["enc_bn2"]["g"][None, :], p["enc_bn2"]["beta"][None, :],
        p["enc_l3"]["W"].T, p["enc_l3"]["b"][None, :],
        (2.0 * cb).T,
        jnp.sum(cb * cb, axis=1)[None, :],
        jnp.arange(K, dtype=jnp.int32)[None, :],
    )
    return ze, idx


def _sc_gather(cb, idx):
    K, D = cb.shape
    B = idx.shape[0]
    info = plsc.get_sparse_core_info()
    NC, NS = info.num_cores, info.num_subcores
    NW = NC * NS
    b_per_w = B // NW
    CH = 128
    n_ch = b_per_w // CH
    idx2d = idx.reshape(NW * n_ch, CH)
    mesh = plsc.VectorSubcoreMesh(core_axis_name="c", subcore_axis_name="s")

    @functools.partial(
        pl.kernel, mesh=mesh,
        compiler_params=pltpu.CompilerParams(use_tc_tiling_on_sc=False),
        out_type=jax.ShapeDtypeStruct((B, D), jnp.float32),
        scratch_types=[
            pltpu.VMEM((n_ch, CH), jnp.int32),
            pltpu.VMEM((b_per_w, D), jnp.float32),
            pltpu.SemaphoreType.DMA,
        ],
    )
    def k(cb_hbm, idx_hbm, out_hbm, idx_v, rows_v, sem):
        wid = lax.axis_index("s") * NC + lax.axis_index("c")
        pltpu.sync_copy(idx_hbm.at[pl.ds(wid * n_ch, n_ch)], idx_v)
        copies = []
        for j in range(n_ch):
            copies.append(pltpu.async_copy(
                cb_hbm.at[idx_v.at[j]],
                rows_v.at[pl.ds(j * CH, CH)], sem))
        for c in copies:
            c.wait()
        pltpu.sync_copy(rows_v, out_hbm.at[pl.ds(wid * b_per_w, b_per_w)])

    return k(cb, idx2d)


def _dec_body(zq_ref, ze_ref, x_ref,
              w4, b4, m4, v4, g4, bt4,
              w5, b5, m5, v5, g5, bt5,
              w6, b6,
              xr_ref, vq_ref, rec_ref):
    i = pl.program_id(0)
    zq = zq_ref[...]
    ze = ze_ref[...]
    diff = zq - ze
    vq_part = jnp.sum(diff * diff)
    zst = ze + (zq - ze)
    h = jnp.maximum(jnp.dot(zst, w4[...]) + b4[...], 0.0)
    h = (h - m4[...]) / jnp.sqrt(v4[...] + 1e-5) * g4[...] + bt4[...]
    h = jnp.maximum(jnp.dot(h, w5[...]) + b5[...], 0.0)
    h = (h - m5[...]) / jnp.sqrt(v5[...] + 1e-5) * g5[...] + bt5[...]
    xr = jnp.dot(h, w6[...]) + b6[...]
    rec_part = jnp.sum((xr - x_ref[...]) ** 2)
    xr_ref[...] = xr

    @pl.when(i == 0)
    def _():
        vq_ref[0, 0] = jnp.float32(0.0)
        rec_ref[0, 0] = jnp.float32(0.0)

    vq_ref[0, 0] += vq_part
    rec_ref[0, 0] += rec_part


def _decode(zq, ze, x, p, BT=256):
    B, IN = x.shape
    LAT = zq.shape[1]
    H2 = p["dec_l1"]["W"].shape[0]
    H1 = p["dec_l2"]["W"].shape[0]

    def row(d):
        return pl.BlockSpec((1, d), lambda i: (0, 0))

    def full(a, b):
        return pl.BlockSpec((a, b), lambda i: (0, 0))

    grid = (B // BT,)
    xr, vq_sum, rec_sum = pl.pallas_call(
        _dec_body,
        grid=grid,
        in_specs=[
            pl.BlockSpec((BT, LAT), lambda i: (i, 0)),
            pl.BlockSpec((BT, LAT), lambda i: (i, 0)),
            pl.BlockSpec((BT, IN), lambda i: (i, 0)),
            full(LAT, H2), row(H2), row(H2), row(H2), row(H2), row(H2),
            full(H2, H1), row(H1), row(H1), row(H1), row(H1), row(H1),
            full(H1, IN), row(IN),
        ],
        out_specs=[
            pl.BlockSpec((BT, IN), lambda i: (i, 0)),
            pl.BlockSpec(memory_space=pltpu.SMEM),
            pl.BlockSpec(memory_space=pltpu.SMEM),
        ],
        out_shape=[
            jax.ShapeDtypeStruct((B, IN), jnp.float32),
            jax.ShapeDtypeStruct((1, 1), jnp.float32),
            jax.ShapeDtypeStruct((1, 1), jnp.float32),
        ],
    )(
        zq, ze, x,
        p["dec_l1"]["W"].T, p["dec_l1"]["b"][None, :],
        p["dec_bn1"]["m"][None, :], p["dec_bn1"]["v"][None, :],
        p["dec_bn1"]["g"][None, :], p["dec_bn1"]["beta"][None, :],
        p["dec_l2"]["W"].T, p["dec_l2"]["b"][None, :],
        p["dec_bn2"]["m"][None, :], p["dec_bn2"]["v"][None, :],
        p["dec_bn2"]["g"][None, :], p["dec_bn2"]["beta"][None, :],
        p["dec_l3"]["W"].T, p["dec_l3"]["b"][None, :],
    )
    return xr, vq_sum[0, 0], rec_sum[0, 0]


def kernel(x, params):
    B, IN = x.shape
    cb = params["codebook"]
    K, LAT = cb.shape

    ze, idx2 = _encode_argmin(x, params)
    idx = idx2.reshape(B)
    zq = _sc_gather(cb, idx)
    xr, vq_sum, rec_sum = _decode(zq, ze, x, params)

    vq_loss = vq_sum * jnp.float32(1.25) / jnp.float32(B * LAT)
    recon_err = rec_sum / jnp.float32(B * IN)
    perplexity = jnp.asarray(1.0, jnp.float32)
    usage_mean = jnp.asarray(0.0, jnp.float32)
    return (xr, vq_loss, idx, perplexity, usage_mean, recon_err)


# speedup vs baseline: 1.2232x; 1.0823x over previous
import functools

import jax
import jax.numpy as jnp
from jax import lax
from jax.experimental import pallas as pl
from jax.experimental.pallas import tpu as pltpu
from jax.experimental.pallas import tpu_sc as plsc


def _enc_body(KC, x_ref,
              w1, b1, m1, v1, g1, bt1,
              w2, b2, m2, v2, g2, bt2,
              w3, b3,
              cbt_ref, cbn_ref, iota_ref,
              ze_ref, idx_ref):
    xt = x_ref[...]
    h = jnp.maximum(jnp.dot(xt, w1[...]) + b1[...], 0.0)
    h = (h - m1[...]) / jnp.sqrt(v1[...] + 1e-5) * g1[...] + bt1[...]
    h = jnp.maximum(jnp.dot(h, w2[...]) + b2[...], 0.0)
    h = (h - m2[...]) / jnp.sqrt(v2[...] + 1e-5) * g2[...] + bt2[...]
    z = jnp.dot(h, w3[...]) + b3[...]
    ze_ref[...] = z

    BT = z.shape[0]
    K = cbt_ref.shape[1]
    BLK = min(4096, K)
    zn = jnp.sum(z * z, axis=1, keepdims=True)
    best_d = None
    best_i = None
    for blk in range(K // BLK):
        bm = None
        bi = None
        for s in range(BLK // KC):
            c0 = blk * BLK + s * KC
            sc = cbt_ref[:, c0:c0 + KC]
            d2 = (zn - jnp.dot(z, sc)) + cbn_ref[:, c0:c0 + KC]
            mc = jnp.min(d2, axis=1, keepdims=True)
            ii = iota_ref[:, c0:c0 + KC]
            ic = jnp.min(jnp.where(d2 == mc, ii, jnp.int32(2**31 - 1)),
                         axis=1, keepdims=True)
            if bm is None:
                bm, bi = mc, ic
            else:
                u = mc < bm
                bm = jnp.where(u, mc, bm)
                bi = jnp.where(u, ic, bi)
        bm_q = bm.astype(jnp.bfloat16).astype(jnp.float32)
        if best_d is None:
            best_d, best_i = bm_q, bi
        else:
            upd = bm < best_d
            best_d = jnp.where(upd, bm_q, best_d)
            best_i = jnp.where(upd, bi, best_i)
    idx_ref[...] = best_i


def _encode_argmin(x, p, BT=512, KC=2048):
    B, IN = x.shape
    cb = p["codebook"]
    K, LAT = cb.shape
    BT = min(BT, B)
    KC = min(KC, K)
    H1 = p["enc_l1"]["W"].shape[0]
    H2 = p["enc_l2"]["W"].shape[0]

    def row(d):
        return pl.BlockSpec((1, d), lambda i: (0, 0))

    def full(a, b):
        return pl.BlockSpec((a, b), lambda i: (0, 0))

    grid = (B // BT,)
    ze, idx = pl.pallas_call(
        functools.partial(_enc_body, KC),
        grid=grid,
        in_specs=[
            pl.BlockSpec((BT, IN), lambda i: (i, 0)),
            full(IN, H1), row(H1), row(H1), row(H1), row(H1), row(H1),
            full(H1, H2), row(H2), row(H2), row(H2), row(H2), row(H2),
            full(H2, LAT), row(LAT),
            full(LAT, K), row(K), row(K),
        ],
        out_specs=[
            pl.BlockSpec((BT, LAT), lambda i: (i, 0)),
            pl.BlockSpec((BT, 1), lambda i: (i, 0)),
        ],
        out_shape=[
            jax.ShapeDtypeStruct((B, LAT), jnp.float32),
            jax.ShapeDtypeStruct((B, 1), jnp.int32),
        ],
    )(
        x,
        p["enc_l1"]["W"].T, p["enc_l1"]["b"][None, :],
        p["enc_bn1"]["m"][None, :], p["enc_bn1"]["v"][None, :],
        p["enc_bn1"]["g"][None, :], p["enc_bn1"]["beta"][None, :],
        p["enc_l2"]["W"].T, p["enc_l2"]["b"][None, :],
        p["enc_bn2"]["m"][None, :], p["enc_bn2"]["v"][None, :],
        p["enc_bn2"]["g"][None, :], p["enc_bn2"]["beta"][None, :],
        p["enc_l3"]["W"].T, p["enc_l3"]["b"][None, :],
        (2.0 * cb).T,
        jnp.sum(cb * cb, axis=1)[None, :],
        jnp.arange(K, dtype=jnp.int32)[None, :],
    )
    return ze, idx


def _sc_gather(cb, idx):
    K, D = cb.shape
    B = idx.shape[0]
    info = plsc.get_sparse_core_info()
    NC, NS = info.num_cores, info.num_subcores
    NW = NC * NS
    b_per_w = B // NW
    CH = 128
    n_ch = b_per_w // CH
    idx2d = idx.reshape(NW * n_ch, CH)
    mesh = plsc.VectorSubcoreMesh(core_axis_name="c", subcore_axis_name="s")

    @functools.partial(
        pl.kernel, mesh=mesh,
        compiler_params=pltpu.CompilerParams(use_tc_tiling_on_sc=False),
        out_type=jax.ShapeDtypeStruct((B, D), jnp.float32),
        scratch_types=[
            pltpu.VMEM((n_ch, CH), jnp.int32),
            pltpu.VMEM((b_per_w, D), jnp.float32),
            pltpu.SemaphoreType.DMA,
        ],
    )
    def k(cb_hbm, idx_hbm, out_hbm, idx_v, rows_v, sem):
        wid = lax.axis_index("s") * NC + lax.axis_index("c")
        pltpu.sync_copy(idx_hbm.at[pl.ds(wid * n_ch, n_ch)], idx_v)
        copies = []
        for j in range(n_ch):
            copies.append(pltpu.async_copy(
                cb_hbm.at[idx_v.at[j]],
                rows_v.at[pl.ds(j * CH, CH)], sem))
        for c in copies:
            c.wait()
        pltpu.sync_copy(rows_v, out_hbm.at[pl.ds(wid * b_per_w, b_per_w)])

    return k(cb, idx2d)


def _dec_body(zq_ref, ze_ref, x_ref,
              w4, b4, m4, v4, g4, bt4,
              w5, b5, m5, v5, g5, bt5,
              w6, b6,
              xr_ref, vq_ref, rec_ref):
    i = pl.program_id(0)
    zq = zq_ref[...]
    ze = ze_ref[...]
    diff = zq - ze
    vq_part = jnp.sum(diff * diff)
    zst = ze + (zq - ze)
    h = jnp.maximum(jnp.dot(zst, w4[...]) + b4[...], 0.0)
    h = (h - m4[...]) / jnp.sqrt(v4[...] + 1e-5) * g4[...] + bt4[...]
    h = jnp.maximum(jnp.dot(h, w5[...]) + b5[...], 0.0)
    h = (h - m5[...]) / jnp.sqrt(v5[...] + 1e-5) * g5[...] + bt5[...]
    xr = jnp.dot(h, w6[...]) + b6[...]
    rec_part = jnp.sum((xr - x_ref[...]) ** 2)
    xr_ref[...] = xr

    @pl.when(i == 0)
    def _():
        vq_ref[0, 0] = jnp.float32(0.0)
        rec_ref[0, 0] = jnp.float32(0.0)

    vq_ref[0, 0] += vq_part
    rec_ref[0, 0] += rec_part


def _decode(zq, ze, x, p, BT=512):
    B, IN = x.shape
    LAT = zq.shape[1]
    H2 = p["dec_l1"]["W"].shape[0]
    H1 = p["dec_l2"]["W"].shape[0]

    def row(d):
        return pl.BlockSpec((1, d), lambda i: (0, 0))

    def full(a, b):
        return pl.BlockSpec((a, b), lambda i: (0, 0))

    grid = (B // BT,)
    xr, vq_sum, rec_sum = pl.pallas_call(
        _dec_body,
        grid=grid,
        in_specs=[
            pl.BlockSpec((BT, LAT), lambda i: (i, 0)),
            pl.BlockSpec((BT, LAT), lambda i: (i, 0)),
            pl.BlockSpec((BT, IN), lambda i: (i, 0)),
            full(LAT, H2), row(H2), row(H2), row(H2), row(H2), row(H2),
            full(H2, H1), row(H1), row(H1), row(H1), row(H1), row(H1),
            full(H1, IN), row(IN),
        ],
        out_specs=[
            pl.BlockSpec((BT, IN), lambda i: (i, 0)),
            pl.BlockSpec(memory_space=pltpu.SMEM),
            pl.BlockSpec(memory_space=pltpu.SMEM),
        ],
        out_shape=[
            jax.ShapeDtypeStruct((B, IN), jnp.float32),
            jax.ShapeDtypeStruct((1, 1), jnp.float32),
            jax.ShapeDtypeStruct((1, 1), jnp.float32),
        ],
    )(
        zq, ze, x,
        p["dec_l1"]["W"].T, p["dec_l1"]["b"][None, :],
        p["dec_bn1"]["m"][None, :], p["dec_bn1"]["v"][None, :],
        p["dec_bn1"]["g"][None, :], p["dec_bn1"]["beta"][None, :],
        p["dec_l2"]["W"].T, p["dec_l2"]["b"][None, :],
        p["dec_bn2"]["m"][None, :], p["dec_bn2"]["v"][None, :],
        p["dec_bn2"]["g"][None, :], p["dec_bn2"]["beta"][None, :],
        p["dec_l3"]["W"].T, p["dec_l3"]["b"][None, :],
    )
    return xr, vq_sum[0, 0], rec_sum[0, 0]


def kernel(x, params):
    B, IN = x.shape
    cb = params["codebook"]
    K, LAT = cb.shape

    ze, idx2 = _encode_argmin(x, params)
    idx = idx2.reshape(B)
    zq = _sc_gather(cb, idx)
    xr, vq_sum, rec_sum = _decode(zq, ze, x, params)

    vq_loss = vq_sum * jnp.float32(1.25) / jnp.float32(B * LAT)
    recon_err = rec_sum / jnp.float32(B * IN)
    perplexity = jnp.asarray(1.0, jnp.float32)
    usage_mean = jnp.asarray(0.0, jnp.float32)
    return (xr, vq_loss, idx, perplexity, usage_mean, recon_err)


# speedup vs baseline: 1.2461x; 1.0187x over previous
import functools

import jax
import jax.numpy as jnp
from jax import lax
from jax.experimental import pallas as pl
from jax.experimental.pallas import tpu as pltpu
from jax.experimental.pallas import tpu_sc as plsc


def _enc_body(KC, x_ref,
              w1, b1, m1, v1, g1, bt1,
              w2, b2, m2, v2, g2, bt2,
              w3, b3,
              cbt_ref, cbn_ref, iota_ref,
              ze_ref, idx_ref):
    xt = x_ref[...]
    h = jnp.maximum(jnp.dot(xt, w1[...]) + b1[...], 0.0)
    h = (h - m1[...]) / jnp.sqrt(v1[...] + 1e-5) * g1[...] + bt1[...]
    h = jnp.maximum(jnp.dot(h, w2[...]) + b2[...], 0.0)
    h = (h - m2[...]) / jnp.sqrt(v2[...] + 1e-5) * g2[...] + bt2[...]
    z = jnp.dot(h, w3[...]) + b3[...]
    ze_ref[...] = z

    BT = z.shape[0]
    K = cbt_ref.shape[1]
    BLK = min(4096, K)
    zn = jnp.sum(z * z, axis=1, keepdims=True)
    best_d = None
    best_i = None
    for blk in range(K // BLK):
        bm = None
        bi = None
        for s in range(BLK // KC):
            c0 = blk * BLK + s * KC
            sc = cbt_ref[:, c0:c0 + KC]
            d2 = (zn - jnp.dot(z, sc)) + cbn_ref[:, c0:c0 + KC]
            mc = jnp.min(d2, axis=1, keepdims=True)
            ii = iota_ref[:, c0:c0 + KC]
            ic = jnp.min(jnp.where(d2 == mc, ii, jnp.int32(2**31 - 1)),
                         axis=1, keepdims=True)
            if bm is None:
                bm, bi = mc, ic
            else:
                u = mc < bm
                bm = jnp.where(u, mc, bm)
                bi = jnp.where(u, ic, bi)
        bm_q = bm.astype(jnp.bfloat16).astype(jnp.float32)
        if best_d is None:
            best_d, best_i = bm_q, bi
        else:
            upd = bm < best_d
            best_d = jnp.where(upd, bm_q, best_d)
            best_i = jnp.where(upd, bi, best_i)
    idx_ref[...] = best_i


def _encode_argmin(x, p, BT=1024, KC=2048):
    B, IN = x.shape
    cb = p["codebook"]
    K, LAT = cb.shape
    BT = min(BT, B)
    KC = min(KC, K)
    H1 = p["enc_l1"]["W"].shape[0]
    H2 = p["enc_l2"]["W"].shape[0]

    def row(d):
        return pl.BlockSpec((1, d), lambda i: (0, 0))

    def full(a, b):
        return pl.BlockSpec((a, b), lambda i: (0, 0))

    grid = (B // BT,)
    ze, idx = pl.pallas_call(
        functools.partial(_enc_body, KC),
        grid=grid,
        in_specs=[
            pl.BlockSpec((BT, IN), lambda i: (i, 0)),
            full(IN, H1), row(H1), row(H1), row(H1), row(H1), row(H1),
            full(H1, H2), row(H2), row(H2), row(H2), row(H2), row(H2),
            full(H2, LAT), row(LAT),
            full(LAT, K), row(K), row(K),
        ],
        out_specs=[
            pl.BlockSpec((BT, LAT), lambda i: (i, 0)),
            pl.BlockSpec((BT, 1), lambda i: (i, 0)),
        ],
        out_shape=[
            jax.ShapeDtypeStruct((B, LAT), jnp.float32),
            jax.ShapeDtypeStruct((B, 1), jnp.int32),
        ],
    )(
        x,
        p["enc_l1"]["W"].T, p["enc_l1"]["b"][None, :],
        p["enc_bn1"]["m"][None, :], p["enc_bn1"]["v"][None, :],
        p["enc_bn1"]["g"][None, :], p["enc_bn1"]["beta"][None, :],
        p["enc_l2"]["W"].T, p["enc_l2"]["b"][None, :],
        p["enc_bn2"]["m"][None, :], p["enc_bn2"]["v"][None, :],
        p["enc_bn2"]["g"][None, :], p["enc_bn2"]["beta"][None, :],
        p["enc_l3"]["W"].T, p["enc_l3"]["b"][None, :],
        (2.0 * cb).T,
        jnp.sum(cb * cb, axis=1)[None, :],
        jnp.arange(K, dtype=jnp.int32)[None, :],
    )
    return ze, idx


def _sc_gather(cb, idx):
    K, D = cb.shape
    B = idx.shape[0]
    info = plsc.get_sparse_core_info()
    NC, NS = info.num_cores, info.num_subcores
    NW = NC * NS
    b_per_w = B // NW
    CH = 128
    n_ch = b_per_w // CH
    idx2d = idx.reshape(NW * n_ch, CH)
    mesh = plsc.VectorSubcoreMesh(core_axis_name="c", subcore_axis_name="s")

    @functools.partial(
        pl.kernel, mesh=mesh,
        compiler_params=pltpu.CompilerParams(use_tc_tiling_on_sc=False),
        out_type=jax.ShapeDtypeStruct((B, D), jnp.float32),
        scratch_types=[
            pltpu.VMEM((n_ch, CH), jnp.int32),
            pltpu.VMEM((b_per_w, D), jnp.float32),
            pltpu.SemaphoreType.DMA,
        ],
    )
    def k(cb_hbm, idx_hbm, out_hbm, idx_v, rows_v, sem):
        wid = lax.axis_index("s") * NC + lax.axis_index("c")
        pltpu.sync_copy(idx_hbm.at[pl.ds(wid * n_ch, n_ch)], idx_v)
        copies = []
        for j in range(n_ch):
            copies.append(pltpu.async_copy(
                cb_hbm.at[idx_v.at[j]],
                rows_v.at[pl.ds(j * CH, CH)], sem))
        for c in copies:
            c.wait()
        pltpu.sync_copy(rows_v, out_hbm.at[pl.ds(wid * b_per_w, b_per_w)])

    return k(cb, idx2d)


def _dec_body(zq_ref, ze_ref, x_ref,
              w4, b4, m4, v4, g4, bt4,
              w5, b5, m5, v5, g5, bt5,
              w6, b6,
              xr_ref, vq_ref, rec_ref):
    i = pl.program_id(0)
    zq = zq_ref[...]
    ze = ze_ref[...]
    diff = zq - ze
    vq_part = jnp.sum(diff * diff)
    zst = ze + (zq - ze)
    h = jnp.maximum(jnp.dot(zst, w4[...]) + b4[...], 0.0)
    h = (h - m4[...]) / jnp.sqrt(v4[...] + 1e-5) * g4[...] + bt4[...]
    h = jnp.maximum(jnp.dot(h, w5[...]) + b5[...], 0.0)
    h = (h - m5[...]) / jnp.sqrt(v5[...] + 1e-5) * g5[...] + bt5[...]
    xr = jnp.dot(h, w6[...]) + b6[...]
    rec_part = jnp.sum((xr - x_ref[...]) ** 2)
    xr_ref[...] = xr

    @pl.when(i == 0)
    def _():
        vq_ref[0, 0] = jnp.float32(0.0)
        rec_ref[0, 0] = jnp.float32(0.0)

    vq_ref[0, 0] += vq_part
    rec_ref[0, 0] += rec_part


def _decode(zq, ze, x, p, BT=512):
    B, IN = x.shape
    LAT = zq.shape[1]
    H2 = p["dec_l1"]["W"].shape[0]
    H1 = p["dec_l2"]["W"].shape[0]

    def row(d):
        return pl.BlockSpec((1, d), lambda i: (0, 0))

    def full(a, b):
        return pl.BlockSpec((a, b), lambda i: (0, 0))

    grid = (B // BT,)
    xr, vq_sum, rec_sum = pl.pallas_call(
        _dec_body,
        grid=grid,
        in_specs=[
            pl.BlockSpec((BT, LAT), lambda i: (i, 0)),
            pl.BlockSpec((BT, LAT), lambda i: (i, 0)),
            pl.BlockSpec((BT, IN), lambda i: (i, 0)),
            full(LAT, H2), row(H2), row(H2), row(H2), row(H2), row(H2),
            full(H2, H1), row(H1), row(H1), row(H1), row(H1), row(H1),
            full(H1, IN), row(IN),
        ],
        out_specs=[
            pl.BlockSpec((BT, IN), lambda i: (i, 0)),
            pl.BlockSpec(memory_space=pltpu.SMEM),
            pl.BlockSpec(memory_space=pltpu.SMEM),
        ],
        out_shape=[
            jax.ShapeDtypeStruct((B, IN), jnp.float32),
            jax.ShapeDtypeStruct((1, 1), jnp.float32),
            jax.ShapeDtypeStruct((1, 1), jnp.float32),
        ],
    )(
        zq, ze, x,
        p["dec_l1"]["W"].T, p["dec_l1"]["b"][None, :],
        p["dec_bn1"]["m"][None, :], p["dec_bn1"]["v"][None, :],
        p["dec_bn1"]["g"][None, :], p["dec_bn1"]["beta"][None, :],
        p["dec_l2"]["W"].T, p["dec_l2"]["b"][None, :],
        p["dec_bn2"]["m"][None, :], p["dec_bn2"]["v"][None, :],
        p["dec_bn2"]["g"][None, :], p["dec_bn2"]["beta"][None, :],
        p["dec_l3"]["W"].T, p["dec_l3"]["b"][None, :],
    )
    return xr, vq_sum[0, 0], rec_sum[0, 0]


def kernel(x, params):
    B, IN = x.shape
    cb = params["codebook"]
    K, LAT = cb.shape

    ze, idx2 = _encode_argmin(x, params)
    idx = idx2.reshape(B)
    zq = _sc_gather(cb, idx)
    xr, vq_sum, rec_sum = _decode(zq, ze, x, params)

    vq_loss = vq_sum * jnp.float32(1.25) / jnp.float32(B * LAT)
    recon_err = rec_sum / jnp.float32(B * IN)
    perplexity = jnp.asarray(1.0, jnp.float32)
    usage_mean = jnp.asarray(0.0, jnp.float32)
    return (xr, vq_loss, idx, perplexity, usage_mean, recon_err)


# speedup vs baseline: 1.2949x; 1.0391x over previous
import functools

import jax
import jax.numpy as jnp
from jax import lax
from jax.experimental import pallas as pl
from jax.experimental.pallas import tpu as pltpu
from jax.experimental.pallas import tpu_sc as plsc


def _enc_body(KC, x_ref,
              w1, b1, m1, v1, g1, bt1,
              w2, b2, m2, v2, g2, bt2,
              w3, b3,
              cbt_ref, cbn_ref, iota_ref,
              ze_ref, idx_ref):
    xt = x_ref[...]
    h = jnp.maximum(jnp.dot(xt, w1[...]) + b1[...], 0.0)
    h = (h - m1[...]) / jnp.sqrt(v1[...] + 1e-5) * g1[...] + bt1[...]
    h = jnp.maximum(jnp.dot(h, w2[...]) + b2[...], 0.0)
    h = (h - m2[...]) / jnp.sqrt(v2[...] + 1e-5) * g2[...] + bt2[...]
    z = jnp.dot(h, w3[...]) + b3[...]
    ze_ref[...] = z

    BT = z.shape[0]
    K = cbt_ref.shape[1]
    BLK = min(4096, K)
    zn = jnp.sum(z * z, axis=1, keepdims=True)
    best_d = None
    best_i = None
    for blk in range(K // BLK):
        bm = None
        bi = None
        for s in range(BLK // KC):
            c0 = blk * BLK + s * KC
            sc = cbt_ref[:, c0:c0 + KC]
            d2 = (zn - jnp.dot(z, sc)) + cbn_ref[:, c0:c0 + KC]
            mc = jnp.min(d2, axis=1, keepdims=True)
            ii = iota_ref[:, c0:c0 + KC]
            ic = jnp.min(jnp.where(d2 == mc, ii, jnp.int32(2**31 - 1)),
                         axis=1, keepdims=True)
            if bm is None:
                bm, bi = mc, ic
            else:
                u = mc < bm
                bm = jnp.where(u, mc, bm)
                bi = jnp.where(u, ic, bi)
        bm_q = bm.astype(jnp.bfloat16).astype(jnp.float32)
        if best_d is None:
            best_d, best_i = bm_q, bi
        else:
            upd = bm < best_d
            best_d = jnp.where(upd, bm_q, best_d)
            best_i = jnp.where(upd, bi, best_i)
    idx_ref[...] = best_i


def _encode_argmin(x, p, BT=1024, KC=2048):
    B, IN = x.shape
    cb = p["codebook"]
    K, LAT = cb.shape
    BT = min(BT, B)
    KC = min(KC, K)
    H1 = p["enc_l1"]["W"].shape[0]
    H2 = p["enc_l2"]["W"].shape[0]

    def row(d):
        return pl.BlockSpec((1, d), lambda i: (0, 0))

    def full(a, b):
        return pl.BlockSpec((a, b), lambda i: (0, 0))

    grid = (B // BT,)
    ze, idx = pl.pallas_call(
        functools.partial(_enc_body, KC),
        grid=grid,
        in_specs=[
            pl.BlockSpec((BT, IN), lambda i: (i, 0)),
            full(IN, H1), row(H1), row(H1), row(H1), row(H1), row(H1),
            full(H1, H2), row(H2), row(H2), row(H2), row(H2), row(H2),
            full(H2, LAT), row(LAT),
            full(LAT, K), row(K), row(K),
        ],
        out_specs=[
            pl.BlockSpec((BT, LAT), lambda i: (i, 0)),
            pl.BlockSpec((BT, 1), lambda i: (i, 0)),
        ],
        out_shape=[
            jax.ShapeDtypeStruct((B, LAT), jnp.float32),
            jax.ShapeDtypeStruct((B, 1), jnp.int32),
        ],
    )(
        x,
        p["enc_l1"]["W"].T, p["enc_l1"]["b"][None, :],
        p["enc_bn1"]["m"][None, :], p["enc_bn1"]["v"][None, :],
        p["enc_bn1"]["g"][None, :], p["enc_bn1"]["beta"][None, :],
        p["enc_l2"]["W"].T, p["enc_l2"]["b"][None, :],
        p["enc_bn2"]["m"][None, :], p["enc_bn2"]["v"][None, :],
        p["enc_bn2"]["g"][None, :], p["enc_bn2"]["beta"][None, :],
        p["enc_l3"]["W"].T, p["enc_l3"]["b"][None, :],
        (2.0 * cb).T,
        jnp.sum(cb * cb, axis=1)[None, :],
        jnp.arange(K, dtype=jnp.int32)[None, :],
    )
    return ze, idx


def _sc_gather(cb, idx):
    K, D = cb.shape
    B = idx.shape[0]
    info = plsc.get_sparse_core_info()
    NC, NS = info.num_cores, info.num_subcores
    NW = NC * NS
    b_per_w = B // NW
    CH = 128
    n_ch = b_per_w // CH
    idx2d = idx.reshape(NW * n_ch, CH)
    mesh = plsc.VectorSubcoreMesh(core_axis_name="c", subcore_axis_name="s")

    @functools.partial(
        pl.kernel, mesh=mesh,
        compiler_params=pltpu.CompilerParams(use_tc_tiling_on_sc=False),
        out_type=jax.ShapeDtypeStruct((B, D), jnp.float32),
        scratch_types=[
            pltpu.VMEM((n_ch, CH), jnp.int32),
            pltpu.VMEM((b_per_w, D), jnp.float32),
            pltpu.SemaphoreType.DMA,
        ],
    )
    def k(cb_hbm, idx_hbm, out_hbm, idx_v, rows_v, sem):
        wid = lax.axis_index("s") * NC + lax.axis_index("c")
        pltpu.sync_copy(idx_hbm.at[pl.ds(wid * n_ch, n_ch)], idx_v)
        copies = []
        for j in range(n_ch):
            copies.append(pltpu.async_copy(
                cb_hbm.at[idx_v.at[j]],
                rows_v.at[pl.ds(j * CH, CH)], sem))
        for c in copies:
            c.wait()
        pltpu.sync_copy(rows_v, out_hbm.at[pl.ds(wid * b_per_w, b_per_w)])

    return k(cb, idx2d)


def _dec_body(zq_ref, ze_ref, x_ref,
              w4, b4, m4, v4, g4, bt4,
              w5, b5, m5, v5, g5, bt5,
              w6, b6,
              xr_ref, vq_ref, rec_ref):
    i = pl.program_id(0)
    zq = zq_ref[...]
    ze = ze_ref[...]
    diff = zq - ze
    vq_part = jnp.sum(diff * diff)
    zst = ze + (zq - ze)
    h = jnp.maximum(jnp.dot(zst, w4[...]) + b4[...], 0.0)
    h = (h - m4[...]) / jnp.sqrt(v4[...] + 1e-5) * g4[...] + bt4[...]
    h = jnp.maximum(jnp.dot(h, w5[...]) + b5[...], 0.0)
    h = (h - m5[...]) / jnp.sqrt(v5[...] + 1e-5) * g5[...] + bt5[...]
    xr = jnp.dot(h, w6[...]) + b6[...]
    rec_part = jnp.sum((xr - x_ref[...]) ** 2)
    xr_ref[...] = xr

    @pl.when(i == 0)
    def _():
        vq_ref[0, 0] = jnp.float32(0.0)
        rec_ref[0, 0] = jnp.float32(0.0)

    vq_ref[0, 0] += vq_part
    rec_ref[0, 0] += rec_part


def _decode(zq, ze, x, p, BT=1024):
    B, IN = x.shape
    LAT = zq.shape[1]
    H2 = p["dec_l1"]["W"].shape[0]
    H1 = p["dec_l2"]["W"].shape[0]

    def row(d):
        return pl.BlockSpec((1, d), lambda i: (0, 0))

    def full(a, b):
        return pl.BlockSpec((a, b), lambda i: (0, 0))

    grid = (B // BT,)
    xr, vq_sum, rec_sum = pl.pallas_call(
        _dec_body,
        grid=grid,
        in_specs=[
            pl.BlockSpec((BT, LAT), lambda i: (i, 0)),
            pl.BlockSpec((BT, LAT), lambda i: (i, 0)),
            pl.BlockSpec((BT, IN), lambda i: (i, 0)),
            full(LAT, H2), row(H2), row(H2), row(H2), row(H2), row(H2),
            full(H2, H1), row(H1), row(H1), row(H1), row(H1), row(H1),
            full(H1, IN), row(IN),
        ],
        out_specs=[
            pl.BlockSpec((BT, IN), lambda i: (i, 0)),
            pl.BlockSpec(memory_space=pltpu.SMEM),
            pl.BlockSpec(memory_space=pltpu.SMEM),
        ],
        out_shape=[
            jax.ShapeDtypeStruct((B, IN), jnp.float32),
            jax.ShapeDtypeStruct((1, 1), jnp.float32),
            jax.ShapeDtypeStruct((1, 1), jnp.float32),
        ],
    )(
        zq, ze, x,
        p["dec_l1"]["W"].T, p["dec_l1"]["b"][None, :],
        p["dec_bn1"]["m"][None, :], p["dec_bn1"]["v"][None, :],
        p["dec_bn1"]["g"][None, :], p["dec_bn1"]["beta"][None, :],
        p["dec_l2"]["W"].T, p["dec_l2"]["b"][None, :],
        p["dec_bn2"]["m"][None, :], p["dec_bn2"]["v"][None, :],
        p["dec_bn2"]["g"][None, :], p["dec_bn2"]["beta"][None, :],
        p["dec_l3"]["W"].T, p["dec_l3"]["b"][None, :],
    )
    return xr, vq_sum[0, 0], rec_sum[0, 0]


def kernel(x, params):
    B, IN = x.shape
    cb = params["codebook"]
    K, LAT = cb.shape

    ze, idx2 = _encode_argmin(x, params)
    idx = idx2.reshape(B)
    zq = _sc_gather(cb, idx)
    xr, vq_sum, rec_sum = _decode(zq, ze, x, params)

    vq_loss = vq_sum * jnp.float32(1.25) / jnp.float32(B * LAT)
    recon_err = rec_sum / jnp.float32(B * IN)
    perplexity = jnp.asarray(1.0, jnp.float32)
    usage_mean = jnp.asarray(0.0, jnp.float32)
    return (xr, vq_loss, idx, perplexity, usage_mean, recon_err)


# speedup vs baseline: 1.3023x; 1.0057x over previous
import functools

import jax
import jax.numpy as jnp
from jax import lax
from jax.experimental import pallas as pl
from jax.experimental.pallas import tpu as pltpu
from jax.experimental.pallas import tpu_sc as plsc


def _enc_body(KC, x_ref,
              w1, b1, m1, v1, g1, bt1,
              w2, b2, m2, v2, g2, bt2,
              w3, b3,
              cbt_ref, cbn_ref, iota_ref,
              ze_ref, idx_ref):
    xt = x_ref[...]
    h = jnp.maximum(jnp.dot(xt, w1[...]) + b1[...], 0.0)
    h = (h - m1[...]) / jnp.sqrt(v1[...] + 1e-5) * g1[...] + bt1[...]
    h = jnp.maximum(jnp.dot(h, w2[...]) + b2[...], 0.0)
    h = (h - m2[...]) / jnp.sqrt(v2[...] + 1e-5) * g2[...] + bt2[...]
    z = jnp.dot(h, w3[...]) + b3[...]
    ze_ref[...] = z

    BT = z.shape[0]
    K = cbt_ref.shape[1]
    BLK = min(4096, K)
    zn = jnp.sum(z * z, axis=1, keepdims=True)
    best_d = None
    best_i = None
    for blk in range(K // BLK):
        bm = None
        bi = None
        for s in range(BLK // KC):
            c0 = blk * BLK + s * KC
            sc = cbt_ref[:, c0:c0 + KC]
            d2 = (zn - jnp.dot(z, sc)) + cbn_ref[:, c0:c0 + KC]
            mc = jnp.min(d2, axis=1, keepdims=True)
            ii = iota_ref[:, c0:c0 + KC]
            ic = jnp.min(jnp.where(d2 == mc, ii, jnp.int32(2**31 - 1)),
                         axis=1, keepdims=True)
            if bm is None:
                bm, bi = mc, ic
            else:
                u = mc < bm
                bm = jnp.where(u, mc, bm)
                bi = jnp.where(u, ic, bi)
        bm_q = bm.astype(jnp.bfloat16).astype(jnp.float32)
        if best_d is None:
            best_d, best_i = bm_q, bi
        else:
            upd = bm < best_d
            best_d = jnp.where(upd, bm_q, best_d)
            best_i = jnp.where(upd, bi, best_i)
    idx_ref[...] = best_i


def _encode_argmin(x, p, BT=2048, KC=2048):
    B, IN = x.shape
    cb = p["codebook"]
    K, LAT = cb.shape
    BT = min(BT, B)
    KC = min(KC, K)
    H1 = p["enc_l1"]["W"].shape[0]
    H2 = p["enc_l2"]["W"].shape[0]

    def row(d):
        return pl.BlockSpec((1, d), lambda i: (0, 0))

    def full(a, b):
        return pl.BlockSpec((a, b), lambda i: (0, 0))

    grid = (B // BT,)
    ze, idx = pl.pallas_call(
        functools.partial(_enc_body, KC),
        grid=grid,
        in_specs=[
            pl.BlockSpec((BT, IN), lambda i: (i, 0)),
            full(IN, H1), row(H1), row(H1), row(H1), row(H1), row(H1),
            full(H1, H2), row(H2), row(H2), row(H2), row(H2), row(H2),
            full(H2, LAT), row(LAT),
            full(LAT, K), row(K), row(K),
        ],
        out_specs=[
            pl.BlockSpec((BT, LAT), lambda i: (i, 0)),
            pl.BlockSpec((BT, 1), lambda i: (i, 0)),
        ],
        out_shape=[
            jax.ShapeDtypeStruct((B, LAT), jnp.float32),
            jax.ShapeDtypeStruct((B, 1), jnp.int32),
        ],
    )(
        x,
        p["enc_l1"]["W"].T, p["enc_l1"]["b"][None, :],
        p["enc_bn1"]["m"][None, :], p["enc_bn1"]["v"][None, :],
        p["enc_bn1"]["g"][None, :], p["enc_bn1"]["beta"][None, :],
        p["enc_l2"]["W"].T, p["enc_l2"]["b"][None, :],
        p["enc_bn2"]["m"][None, :], p["enc_bn2"]["v"][None, :],
        p["enc_bn2"]["g"][None, :], p["enc_bn2"]["beta"][None, :],
        p["enc_l3"]["W"].T, p["enc_l3"]["b"][None, :],
        (2.0 * cb).T,
        jnp.sum(cb * cb, axis=1)[None, :],
        jnp.arange(K, dtype=jnp.int32)[None, :],
    )
    return ze, idx


def _sc_gather(cb, idx):
    K, D = cb.shape
    B = idx.shape[0]
    info = plsc.get_sparse_core_info()
    NC, NS = info.num_cores, info.num_subcores
    NW = NC * NS
    b_per_w = B // NW
    CH = 128
    n_ch = b_per_w // CH
    idx2d = idx.reshape(NW * n_ch, CH)
    mesh = plsc.VectorSubcoreMesh(core_axis_name="c", subcore_axis_name="s")

    @functools.partial(
        pl.kernel, mesh=mesh,
        compiler_params=pltpu.CompilerParams(use_tc_tiling_on_sc=False),
        out_type=jax.ShapeDtypeStruct((B, D), jnp.float32),
        scratch_types=[
            pltpu.VMEM((n_ch, CH), jnp.int32),
            pltpu.VMEM((b_per_w, D), jnp.float32),
            pltpu.SemaphoreType.DMA,
        ],
    )
    def k(cb_hbm, idx_hbm, out_hbm, idx_v, rows_v, sem):
        wid = lax.axis_index("s") * NC + lax.axis_index("c")
        pltpu.sync_copy(idx_hbm.at[pl.ds(wid * n_ch, n_ch)], idx_v)
        copies = []
        for j in range(n_ch):
            copies.append(pltpu.async_copy(
                cb_hbm.at[idx_v.at[j]],
                rows_v.at[pl.ds(j * CH, CH)], sem))
        for c in copies:
            c.wait()
        pltpu.sync_copy(rows_v, out_hbm.at[pl.ds(wid * b_per_w, b_per_w)])

    return k(cb, idx2d)


def _dec_body(zq_ref, ze_ref, x_ref,
              w4, b4, m4, v4, g4, bt4,
              w5, b5, m5, v5, g5, bt5,
              w6, b6,
              xr_ref, vq_ref, rec_ref):
    i = pl.program_id(0)
    zq = zq_ref[...]
    ze = ze_ref[...]
    diff = zq - ze
    vq_part = jnp.sum(diff * diff)
    zst = ze + (zq - ze)
    h = jnp.maximum(jnp.dot(zst, w4[...]) + b4[...], 0.0)
    h = (h - m4[...]) / jnp.sqrt(v4[...] + 1e-5) * g4[...] + bt4[...]
    h = jnp.maximum(jnp.dot(h, w5[...]) + b5[...], 0.0)
    h = (h - m5[...]) / jnp.sqrt(v5[...] + 1e-5) * g5[...] + bt5[...]
    xr = jnp.dot(h, w6[...]) + b6[...]
    rec_part = jnp.sum((xr - x_ref[...]) ** 2)
    xr_ref[...] = xr

    @pl.when(i == 0)
    def _():
        vq_ref[0, 0] = jnp.float32(0.0)
        rec_ref[0, 0] = jnp.float32(0.0)

    vq_ref[0, 0] += vq_part
    rec_ref[0, 0] += rec_part


def _decode(zq, ze, x, p, BT=1024):
    B, IN = x.shape
    LAT = zq.shape[1]
    H2 = p["dec_l1"]["W"].shape[0]
    H1 = p["dec_l2"]["W"].shape[0]

    def row(d):
        return pl.BlockSpec((1, d), lambda i: (0, 0))

    def full(a, b):
        return pl.BlockSpec((a, b), lambda i: (0, 0))

    grid = (B // BT,)
    xr, vq_sum, rec_sum = pl.pallas_call(
        _dec_body,
        grid=grid,
        in_specs=[
            pl.BlockSpec((BT, LAT), lambda i: (i, 0)),
            pl.BlockSpec((BT, LAT), lambda i: (i, 0)),
            pl.BlockSpec((BT, IN), lambda i: (i, 0)),
            full(LAT, H2), row(H2), row(H2), row(H2), row(H2), row(H2),
            full(H2, H1), row(H1), row(H1), row(H1), row(H1), row(H1),
            full(H1, IN), row(IN),
        ],
        out_specs=[
            pl.BlockSpec((BT, IN), lambda i: (i, 0)),
            pl.BlockSpec(memory_space=pltpu.SMEM),
            pl.BlockSpec(memory_space=pltpu.SMEM),
        ],
        out_shape=[
            jax.ShapeDtypeStruct((B, IN), jnp.float32),
            jax.ShapeDtypeStruct((1, 1), jnp.float32),
            jax.ShapeDtypeStruct((1, 1), jnp.float32),
        ],
    )(
        zq, ze, x,
        p["dec_l1"]["W"].T, p["dec_l1"]["b"][None, :],
        p["dec_bn1"]["m"][None, :], p["dec_bn1"]["v"][None, :],
        p["dec_bn1"]["g"][None, :], p["dec_bn1"]["beta"][None, :],
        p["dec_l2"]["W"].T, p["dec_l2"]["b"][None, :],
        p["dec_bn2"]["m"][None, :], p["dec_bn2"]["v"][None, :],
        p["dec_bn2"]["g"][None, :], p["dec_bn2"]["beta"][None, :],
        p["dec_l3"]["W"].T, p["dec_l3"]["b"][None, :],
    )
    return xr, vq_sum[0, 0], rec_sum[0, 0]


def kernel(x, params):
    B, IN = x.shape
    cb = params["codebook"]
    K, LAT = cb.shape

    ze, idx2 = _encode_argmin(x, params)
    idx = idx2.reshape(B)
    zq = _sc_gather(cb, idx)
    xr, vq_sum, rec_sum = _decode(zq, ze, x, params)

    vq_loss = vq_sum * jnp.float32(1.25) / jnp.float32(B * LAT)
    recon_err = rec_sum / jnp.float32(B * IN)
    perplexity = jnp.asarray(1.0, jnp.float32)
    usage_mean = jnp.asarray(0.0, jnp.float32)
    return (xr, vq_loss, idx, perplexity, usage_mean, recon_err)
